# baseline (speedup 1.0000x reference)
import functools

import jax
import jax.numpy as jnp
from jax import lax
from jax.experimental import pallas as pl
from jax.experimental.pallas import tpu as pltpu
from jax.experimental.pallas import tpu_sc as plsc

_LR = 0.01
_W_MIN = 0.0
_W_MAX = 1.0

_N_POST = 16384
_N_PRE = 4096
_NW = 32
_ROWS_W = _N_POST // _NW
_CR = 4
_NCHUNK = _ROWS_W // _CR
_NBUF = 2
_L = 16


def _sc_body(w_hbm, pre_hbm, spost_hbm, out_hbm,
             pre_v, spost_v, in_buf, out_buf, in_sem, out_sem):
    cid = lax.axis_index("c")
    sid = lax.axis_index("s")
    wid = sid * 2 + cid
    base = wid * _ROWS_W

    pltpu.sync_copy(pre_hbm, pre_v)
    pltpu.sync_copy(spost_hbm.at[pl.ds(base, _ROWS_W)],
                    spost_v.at[pl.ds(0, _ROWS_W)])

    def start_in(chunk, b):
        pltpu.async_copy(
            w_hbm.at[pl.ds(base + chunk * _CR, _CR)], in_buf.at[b], in_sem.at[b])

    def wait_in(b):
        pltpu.make_async_copy(
            w_hbm.at[pl.ds(base, _CR)], in_buf.at[b], in_sem.at[b]).wait()

    def start_out(chunk, b):
        pltpu.async_copy(
            out_buf.at[b], out_hbm.at[pl.ds(base + chunk * _CR, _CR)],
            out_sem.at[b])

    def wait_out(b):
        pltpu.make_async_copy(
            out_buf.at[b], out_hbm.at[pl.ds(base, _CR)], out_sem.at[b]).wait()

    def compute(chunk, b):
        iref = in_buf.at[b]
        oref = out_buf.at[b]
        sv = spost_v[pl.ds(chunk * _CR, _L)]
        svecs = [jnp.full((_L,), sv[r], jnp.float32) for r in range(_CR)]

        @plsc.parallel_loop(0, _N_PRE, _L, unroll=8)
        def _(j):
            sl = pl.ds(j, _L)
            p = pre_v[sl]
            for r in range(_CR):
                w = iref[r, sl]
                oref[r, sl] = jnp.clip(w + svecs[r] * p, _W_MIN, _W_MAX)

    for b in range(_NBUF):
        start_in(b, b)
    for b in range(_NBUF):
        wait_in(b)
        compute(b, b)
        start_out(b, b)
        start_in(b + _NBUF, b)

    def gbody(g2, _):
        for b in range(_NBUF):
            g = g2 * _NBUF + b
            wait_in(b)
            wait_out(b)
            compute(g, b)
            start_out(g, b)
            nxt = lax.rem(g + _NBUF, _NCHUNK)
            start_in(nxt, b)
        return 0

    lax.fori_loop(1, _NCHUNK // _NBUF, gbody, 0)

    for b in range(_NBUF):
        wait_in(b)
        wait_out(b)


def _kernel_sc(weights, pre, post):
    spost = _LR * post
    mesh = plsc.VectorSubcoreMesh(core_axis_name="c", subcore_axis_name="s")
    f = functools.partial(
        pl.kernel,
        mesh=mesh,
        out_type=jax.ShapeDtypeStruct((_N_POST, _N_PRE), jnp.float32),
        scratch_types=[
            pltpu.VMEM((_N_PRE,), jnp.float32),
            pltpu.VMEM((_ROWS_W + _L,), jnp.float32),
            pltpu.VMEM((_NBUF, _CR, _N_PRE), jnp.float32),
            pltpu.VMEM((_NBUF, _CR, _N_PRE), jnp.float32),
            pltpu.SemaphoreType.DMA((_NBUF,)),
            pltpu.SemaphoreType.DMA((_NBUF,)),
        ],
    )(_sc_body)
    return f(weights, pre, spost)


_BR = 512
_RB = 8


def _update_block(w_ref, pre_ref, post_ref, out_ref):
    pre_row = pre_ref[...]
    for i in range(_BR // _RB):
        sl = pl.ds(i * _RB, _RB)
        dw = post_ref[sl, :] * pre_row
        out_ref[sl, :] = jnp.clip(w_ref[sl, :] + dw, _W_MIN, _W_MAX)


def _kernel_tc(weights, pre, post):
    n_post, n_pre = weights.shape
    pre2 = pre.reshape(1, n_pre)
    post2 = (_LR * post).reshape(n_post, 1)
    grid = (n_post // _BR,)
    return pl.pallas_call(
        _update_block,
        grid=grid,
        in_specs=[
            pl.BlockSpec((_BR, n_pre), lambda i: (i, 0)),
            pl.BlockSpec((1, n_pre), lambda i: (0, 0)),
            pl.BlockSpec((_BR, 1), lambda i: (i, 0)),
        ],
        out_specs=pl.BlockSpec((_BR, n_pre), lambda i: (i, 0)),
        out_shape=jax.ShapeDtypeStruct((n_post, n_pre), weights.dtype),
    )(weights, pre2, post2)


_MC_ROWS = 128
_MC_N = _N_POST // _MC_ROWS
_MC_NBUF = 8
_MC_RB = 8


def _tc_manual_body(w_hbm, spost_hbm, pre_ref, out_hbm,
                    in_buf, out_buf, spost_v, in_sem, out_sem, s_sem):
    pltpu.make_async_copy(spost_hbm, spost_v, s_sem).start()
    pltpu.make_async_copy(spost_hbm, spost_v, s_sem).wait()
    pre_row = pre_ref[...]

    def start_in(c, b):
        pltpu.make_async_copy(
            w_hbm.at[pl.ds(c * _MC_ROWS, _MC_ROWS), :], in_buf.at[b],
            in_sem.at[b]).start()

    def wait_in(b):
        pltpu.make_async_copy(
            w_hbm.at[pl.ds(0, _MC_ROWS), :], in_buf.at[b], in_sem.at[b]).wait()

    def start_out(c, b):
        pltpu.make_async_copy(
            out_buf.at[b], out_hbm.at[pl.ds(c * _MC_ROWS, _MC_ROWS), :],
            out_sem.at[b]).start()

    def wait_out(b):
        pltpu.make_async_copy(
            out_buf.at[b], out_hbm.at[pl.ds(0, _MC_ROWS), :],
            out_sem.at[b]).wait()

    def compute(c, b):
        for i in range(_MC_ROWS // _MC_RB):
            sl = pl.ds(i * _MC_RB, _MC_RB)
            s = spost_v[pl.ds(c * _MC_ROWS + i * _MC_RB, _MC_RB), :]
            out_buf[b, sl, :] = jnp.clip(
                in_buf[b, sl, :] + s * pre_row, _W_MIN, _W_MAX)

    for b in range(_MC_NBUF):
        start_in(b, b)
    for b in range(_MC_NBUF):
        wait_in(b)
        compute(b, b)
        start_out(b, b)
        start_in(b + _MC_NBUF, b)

    def gbody(g2, _):
        for b in range(_MC_NBUF):
            c = g2 * _MC_NBUF + b
            wait_in(b)
            wait_out(b)
            compute(c, b)
            start_out(c, b)
            start_in(c + _MC_NBUF, b)
        return 0

    lax.fori_loop(1, _MC_N // _MC_NBUF - 1, gbody, 0)

    for b in range(_MC_NBUF):
        c = _MC_N - _MC_NBUF + b
        wait_in(b)
        wait_out(b)
        compute(c, b)
        start_out(c, b)
    for b in range(_MC_NBUF):
        wait_out(b)


def _kernel_tc_manual(weights, pre, post):
    pre2 = pre.reshape(1, _N_PRE)
    spost = (_LR * post).reshape(_N_POST, 1)
    return pl.pallas_call(
        _tc_manual_body,
        in_specs=[
            pl.BlockSpec(memory_space=pl.ANY),
            pl.BlockSpec(memory_space=pl.ANY),
            pl.BlockSpec(memory_space=pltpu.VMEM),
        ],
        out_specs=pl.BlockSpec(memory_space=pl.ANY),
        out_shape=jax.ShapeDtypeStruct((_N_POST, _N_PRE), jnp.float32),
        scratch_shapes=[
            pltpu.VMEM((_MC_NBUF, _MC_ROWS, _N_PRE), jnp.float32),
            pltpu.VMEM((_MC_NBUF, _MC_ROWS, _N_PRE), jnp.float32),
            pltpu.VMEM((_N_POST, 1), jnp.float32),
            pltpu.SemaphoreType.DMA((_MC_NBUF,)),
            pltpu.SemaphoreType.DMA((_MC_NBUF,)),
            pltpu.SemaphoreType.DMA,
        ],
    )(weights, spost, pre2)


def kernel(weights, pre, post):
    return _kernel_tc_manual(weights, pre, post)


# speedup vs baseline: 1.0542x; 1.0542x over previous
import functools

import jax
import jax.numpy as jnp
from jax import lax
from jax.experimental import pallas as pl
from jax.experimental.pallas import tpu as pltpu
from jax.experimental.pallas import tpu_sc as plsc

_LR = 0.01
_W_MIN = 0.0
_W_MAX = 1.0

_N_POST = 16384
_N_PRE = 4096
_NW = 32
_ROWS_W = _N_POST // _NW
_CR = 4
_NCHUNK = _ROWS_W // _CR
_NBUF = 2
_L = 16


def _sc_body(w_hbm, pre_hbm, spost_hbm, out_hbm,
             pre_v, spost_v, in_buf, out_buf, in_sem, out_sem):
    cid = lax.axis_index("c")
    sid = lax.axis_index("s")
    wid = sid * 2 + cid
    base = wid * _ROWS_W

    pltpu.sync_copy(pre_hbm, pre_v)
    pltpu.sync_copy(spost_hbm.at[pl.ds(base, _ROWS_W)],
                    spost_v.at[pl.ds(0, _ROWS_W)])

    def start_in(chunk, b):
        pltpu.async_copy(
            w_hbm.at[pl.ds(base + chunk * _CR, _CR)], in_buf.at[b], in_sem.at[b])

    def wait_in(b):
        pltpu.make_async_copy(
            w_hbm.at[pl.ds(base, _CR)], in_buf.at[b], in_sem.at[b]).wait()

    def start_out(chunk, b):
        pltpu.async_copy(
            out_buf.at[b], out_hbm.at[pl.ds(base + chunk * _CR, _CR)],
            out_sem.at[b])

    def wait_out(b):
        pltpu.make_async_copy(
            out_buf.at[b], out_hbm.at[pl.ds(base, _CR)], out_sem.at[b]).wait()

    def compute(chunk, b):
        iref = in_buf.at[b]
        oref = out_buf.at[b]
        sv = spost_v[pl.ds(chunk * _CR, _L)]
        svecs = [jnp.full((_L,), sv[r], jnp.float32) for r in range(_CR)]

        @plsc.parallel_loop(0, _N_PRE, _L, unroll=8)
        def _(j):
            sl = pl.ds(j, _L)
            p = pre_v[sl]
            for r in range(_CR):
                w = iref[r, sl]
                oref[r, sl] = jnp.clip(w + svecs[r] * p, _W_MIN, _W_MAX)

    for b in range(_NBUF):
        start_in(b, b)
    for b in range(_NBUF):
        wait_in(b)
        compute(b, b)
        start_out(b, b)
        start_in(b + _NBUF, b)

    def gbody(g2, _):
        for b in range(_NBUF):
            g = g2 * _NBUF + b
            wait_in(b)
            wait_out(b)
            compute(g, b)
            start_out(g, b)
            nxt = lax.rem(g + _NBUF, _NCHUNK)
            start_in(nxt, b)
        return 0

    lax.fori_loop(1, _NCHUNK // _NBUF, gbody, 0)

    for b in range(_NBUF):
        wait_in(b)
        wait_out(b)


def _kernel_sc(weights, pre, post):
    spost = _LR * post
    mesh = plsc.VectorSubcoreMesh(core_axis_name="c", subcore_axis_name="s")
    f = functools.partial(
        pl.kernel,
        mesh=mesh,
        out_type=jax.ShapeDtypeStruct((_N_POST, _N_PRE), jnp.float32),
        scratch_types=[
            pltpu.VMEM((_N_PRE,), jnp.float32),
            pltpu.VMEM((_ROWS_W + _L,), jnp.float32),
            pltpu.VMEM((_NBUF, _CR, _N_PRE), jnp.float32),
            pltpu.VMEM((_NBUF, _CR, _N_PRE), jnp.float32),
            pltpu.SemaphoreType.DMA((_NBUF,)),
            pltpu.SemaphoreType.DMA((_NBUF,)),
        ],
    )(_sc_body)
    return f(weights, pre, spost)


_BR = 512
_RB = 8


def _update_block(w_ref, pre_ref, post_ref, out_ref):
    pre_row = pre_ref[...]
    for i in range(_BR // _RB):
        sl = pl.ds(i * _RB, _RB)
        dw = post_ref[sl, :] * pre_row
        out_ref[sl, :] = jnp.clip(w_ref[sl, :] + dw, _W_MIN, _W_MAX)


def _kernel_tc(weights, pre, post):
    n_post, n_pre = weights.shape
    pre2 = pre.reshape(1, n_pre)
    post2 = (_LR * post).reshape(n_post, 1)
    grid = (n_post // _BR,)
    return pl.pallas_call(
        _update_block,
        grid=grid,
        in_specs=[
            pl.BlockSpec((_BR, n_pre), lambda i: (i, 0)),
            pl.BlockSpec((1, n_pre), lambda i: (0, 0)),
            pl.BlockSpec((_BR, 1), lambda i: (i, 0)),
        ],
        out_specs=pl.BlockSpec((_BR, n_pre), lambda i: (i, 0)),
        out_shape=jax.ShapeDtypeStruct((n_post, n_pre), weights.dtype),
    )(weights, pre2, post2)


_MC_ROWS = 256
_MC_N = _N_POST // _MC_ROWS
_MC_NBUF = 4
_MC_RB = 8


def _tc_manual_body(w_hbm, post_ref, pre_ref, out_hbm,
                    in_buf, out_buf, spost_v, in_sem, out_sem):
    spost_v[...] = jnp.transpose(post_ref[...] * _LR)
    pre_row = pre_ref[...]

    def start_in(c, b):
        pltpu.make_async_copy(
            w_hbm.at[pl.ds(c * _MC_ROWS, _MC_ROWS), :], in_buf.at[b],
            in_sem.at[b]).start()

    def wait_in(b):
        pltpu.make_async_copy(
            w_hbm.at[pl.ds(0, _MC_ROWS), :], in_buf.at[b], in_sem.at[b]).wait()

    def start_out(c, b):
        pltpu.make_async_copy(
            out_buf.at[b], out_hbm.at[pl.ds(c * _MC_ROWS, _MC_ROWS), :],
            out_sem.at[b]).start()

    def wait_out(b):
        pltpu.make_async_copy(
            out_buf.at[b], out_hbm.at[pl.ds(0, _MC_ROWS), :],
            out_sem.at[b]).wait()

    def compute(c, b):
        for i in range(_MC_ROWS // _MC_RB):
            sl = pl.ds(i * _MC_RB, _MC_RB)
            s = spost_v[pl.ds(c * _MC_ROWS + i * _MC_RB, _MC_RB), :]
            out_buf[b, sl, :] = jnp.clip(
                in_buf[b, sl, :] + s * pre_row, _W_MIN, _W_MAX)

    for b in range(_MC_NBUF):
        start_in(b, b)
    for b in range(_MC_NBUF):
        wait_in(b)
        compute(b, b)
        start_out(b, b)
        start_in(b + _MC_NBUF, b)

    def gbody(g2, _):
        for b in range(_MC_NBUF):
            c = g2 * _MC_NBUF + b
            wait_in(b)
            wait_out(b)
            compute(c, b)
            start_out(c, b)
            start_in(c + _MC_NBUF, b)
        return 0

    lax.fori_loop(1, _MC_N // _MC_NBUF - 1, gbody, 0)

    for b in range(_MC_NBUF):
        c = _MC_N - _MC_NBUF + b
        wait_in(b)
        wait_out(b)
        compute(c, b)
        start_out(c, b)
    for b in range(_MC_NBUF):
        wait_out(b)


def _kernel_tc_manual(weights, pre, post):
    pre2 = pre.reshape(1, _N_PRE)
    postr = post.reshape(1, _N_POST)
    return pl.pallas_call(
        _tc_manual_body,
        in_specs=[
            pl.BlockSpec(memory_space=pl.ANY),
            pl.BlockSpec(memory_space=pltpu.VMEM),
            pl.BlockSpec(memory_space=pltpu.VMEM),
        ],
        out_specs=pl.BlockSpec(memory_space=pl.ANY),
        out_shape=jax.ShapeDtypeStruct((_N_POST, _N_PRE), jnp.float32),
        scratch_shapes=[
            pltpu.VMEM((_MC_NBUF, _MC_ROWS, _N_PRE), jnp.float32),
            pltpu.VMEM((_MC_NBUF, _MC_ROWS, _N_PRE), jnp.float32),
            pltpu.VMEM((_N_POST, 1), jnp.float32),
            pltpu.SemaphoreType.DMA((_MC_NBUF,)),
            pltpu.SemaphoreType.DMA((_MC_NBUF,)),
        ],
    )(weights, postr, pre2)


def kernel(weights, pre, post):
    return _kernel_tc_manual(weights, pre, post)
